# baseline (speedup 1.0000x reference)
import functools

import jax
import jax.numpy as jnp
from jax import lax
from jax.experimental import pallas as pl
from jax.experimental.pallas import tpu as pltpu
from jax.experimental.pallas import tpu_sc as plsc

N = 10000
G = 64
NP = 10112
DUMMY = 10008
NSUB = 16
ZCH = 632
BN = 1000
NB = N // BN

def _make_sc_agg(rt: int, split: bool, er: int):
    mesh = plsc.VectorSubcoreMesh(core_axis_name="c", subcore_axis_name="s")

    @functools.partial(
        pl.kernel,
        out_type=jax.ShapeDtypeStruct((2, N, 128), jnp.float32),
        mesh=mesh,
        scratch_types=[
            pltpu.VMEM((20, 2, 64), jnp.int32),
            pltpu.VMEM((20, 2, 64), jnp.int32),
            [pltpu.VMEM((64, 128), jnp.float32) for _ in range(4)],
            pltpu.VMEM_SHARED((NP, 128), jnp.float32),
            [pltpu.SemaphoreType.DMA for _ in range(4)],
            [pltpu.SemaphoreType.DMA for _ in range(4)],
        ],
    )
    def sc_agg(h_hbm, src_hbm, dst_hbm, out_hbm, src_v, dst_v,
               rows, acc, gsem, ssem):
        c = lax.axis_index("c")
        s = lax.axis_index("s")
        if split:
            base = s * rt
        else:
            base = (c * NSUB + s) * rt

        def zrow(i, carry):
            for j in range(8):
                rows[0][i, pl.ds(j * 16, 16)] = jnp.zeros((16,), jnp.float32)
            return carry

        lax.fori_loop(0, 64, zrow, 0)
        zoff = s * ZCH
        for k in range(9):
            pltpu.sync_copy(rows[0].at[pl.ds(0, 64)],
                            acc.at[pl.ds(zoff + 64 * k, 64)])
        pltpu.sync_copy(rows[0].at[pl.ds(0, ZCH - 576)],
                        acc.at[pl.ds(zoff + 576, ZCH - 576)])
        plsc.subcore_barrier()

        def chunk(ic, carry):
            cb = base + ic * 20
            if split:
                pltpu.sync_copy(src_hbm.at[c, pl.ds(cb, 20)], src_v)
            else:
                pltpu.sync_copy(src_hbm.at[pl.ds(cb, 20)], src_v)
            pltpu.sync_copy(dst_hbm.at[pl.ds(cb, 20)], dst_v)
            pltpu.async_copy(h_hbm.at[src_v.at[0, 0]], rows[0], gsem[0])
            pltpu.async_copy(h_hbm.at[src_v.at[0, 1]], rows[1], gsem[1])

            def quad(q, carry2):
                for k in range(4):
                    j = 4 * q + k
                    pltpu.make_async_copy(h_hbm.at[src_v.at[j // 2, j % 2]],
                                          rows[k], gsem[k]).wait()
                    pltpu.async_copy(rows[k], acc.at[dst_v.at[j // 2, j % 2]],
                                     ssem[k], add=True)
                    kn = (k + 2) % 4
                    jn = j + 2

                    @pl.when(jn < 40)
                    def _():
                        @pl.when(jn >= 4)
                        def _():
                            pltpu.make_async_copy(
                                rows[kn], acc.at[dst_v.at[0, 0]],
                                ssem[kn]).wait()

                        pltpu.async_copy(h_hbm.at[src_v.at[jn // 2, jn % 2]],
                                         rows[kn], gsem[kn])
                return carry2

            lax.fori_loop(0, 10, quad, 0)
            for k in range(4):
                pltpu.make_async_copy(rows[k], acc.at[dst_v.at[0, 0]],
                                      ssem[k]).wait()
            return carry

        lax.fori_loop(0, rt // 20, chunk, 0)
        plsc.subcore_barrier()

        ooff = s * ZCH
        for k in range(4):
            pltpu.sync_copy(acc.at[pl.ds(ooff + 128 * k, 128)],
                            out_hbm.at[c, pl.ds(ooff + 128 * k, 128)])

        @pl.when(s < NSUB - 1)
        def _():
            pltpu.sync_copy(acc.at[pl.ds(ooff + 512, ZCH - 512)],
                            out_hbm.at[c, pl.ds(ooff + 512, ZCH - 512)])

        @pl.when(s == NSUB - 1)
        def _():
            rem = N - (NSUB - 1) * ZCH - 512
            pltpu.sync_copy(acc.at[pl.ds(ooff + 512, rem)],
                            out_hbm.at[c, pl.ds(ooff + 512, rem)])

    return sc_agg


_SC_AGG_CACHE = {}


def _sc_agg(table, src2, dst2, split):
    er = dst2.shape[0]
    key = (split, er)
    if key not in _SC_AGG_CACHE:
        rt = er // NSUB if split else er // (2 * NSUB)
        _SC_AGG_CACHE[key] = _make_sc_agg(rt, split, er)
    return _SC_AGG_CACHE[key](table, src2, dst2)


def _mlp_body(split, eps_ref, h_ref, agg_ref, w1_ref, b1_ref, w2_ref, b2_ref,
              h2_ref, ssum_ref, ssq_ref):
    i = pl.program_id(0)
    if split:
        h = jnp.concatenate([h_ref[0], h_ref[1]], axis=1)
        a = jnp.concatenate([agg_ref[0], agg_ref[1]], axis=1)
    else:
        h = h_ref[...]
        a = agg_ref[0] + agg_ref[1]
    z = (1.0 + eps_ref[0]) * h + a
    t = jnp.maximum(
        jnp.dot(z, w1_ref[...], preferred_element_type=jnp.float32)
        + b1_ref[...], 0.0)
    h2 = jnp.maximum(
        jnp.dot(t, w2_ref[...], preferred_element_type=jnp.float32)
        + b2_ref[...], 0.0)
    h2_ref[...] = h2
    ssum = jnp.sum(h2, axis=0, keepdims=True)
    ssq = jnp.sum(h2 * h2, axis=0, keepdims=True)

    @pl.when(i == 0)
    def _():
        ssum_ref[...] = ssum
        ssq_ref[...] = ssq

    @pl.when(i > 0)
    def _():
        ssum_ref[...] += ssum
        ssq_ref[...] += ssq


def _make_mlp(split, din, hdim):
    dh = din // 2
    if split:
        h_spec = pl.BlockSpec((2, BN, dh), lambda i: (0, i, 0))
    else:
        h_spec = pl.BlockSpec((BN, din), lambda i: (i, 0))
    return pl.pallas_call(
        functools.partial(_mlp_body, split),
        grid=(NB,),
        in_specs=[
            pl.BlockSpec(memory_space=pltpu.SMEM),
            h_spec,
            pl.BlockSpec((2, BN, dh if split else din), lambda i: (0, i, 0)),
            pl.BlockSpec((din, hdim), lambda i: (0, 0)),
            pl.BlockSpec((1, hdim), lambda i: (0, 0)),
            pl.BlockSpec((hdim, hdim), lambda i: (0, 0)),
            pl.BlockSpec((1, hdim), lambda i: (0, 0)),
        ],
        out_specs=[
            pl.BlockSpec((BN, hdim), lambda i: (i, 0)),
            pl.BlockSpec((1, hdim), lambda i: (0, 0)),
            pl.BlockSpec((1, hdim), lambda i: (0, 0)),
        ],
        out_shape=[
            jax.ShapeDtypeStruct((N, hdim), jnp.float32),
            jax.ShapeDtypeStruct((1, hdim), jnp.float32),
            jax.ShapeDtypeStruct((1, hdim), jnp.float32),
        ],
    )


def _bn_split_body(h2_ref, ssum_ref, ssq_ref, g_ref, b_ref, out_ref):
    mean = ssum_ref[...] * (1.0 / N)
    var = ssq_ref[...] * (1.0 / N) - mean * mean
    scale = g_ref[...] / jnp.sqrt(var + 1e-5)
    hn = (h2_ref[...] - mean) * scale + b_ref[...]
    dh = hn.shape[1] // 2
    out_ref[0] = hn[:, :dh]
    out_ref[1] = hn[:, dh:]


def _make_bn_split(hdim):
    dh = hdim // 2
    return pl.pallas_call(
        _bn_split_body,
        grid=(NB,),
        in_specs=[
            pl.BlockSpec((BN, hdim), lambda i: (i, 0)),
            pl.BlockSpec((1, hdim), lambda i: (0, 0)),
            pl.BlockSpec((1, hdim), lambda i: (0, 0)),
            pl.BlockSpec((1, hdim), lambda i: (0, 0)),
            pl.BlockSpec((1, hdim), lambda i: (0, 0)),
        ],
        out_specs=[pl.BlockSpec((2, BN, dh), lambda i: (0, i, 0))],
        out_shape=[jax.ShapeDtypeStruct((2, N, dh), jnp.float32)],
    )


def _pool_body(h2_ref, ssum_ref, ssq_ref, g_ref, b_ref, bat_ref,
               w1_ref, b1_ref, w2_ref, b2_ref, out_ref, pool_acc, cnt_acc):
    i = pl.program_id(0)
    mean = ssum_ref[...] * (1.0 / N)
    var = ssq_ref[...] * (1.0 / N) - mean * mean
    scale = g_ref[...] / jnp.sqrt(var + 1e-5)
    hn = (h2_ref[...] - mean) * scale + b_ref[...]
    bcol = bat_ref[0]
    onehot = (bcol == lax.broadcasted_iota(jnp.int32, (BN, G), 1)
              ).astype(jnp.float32)
    dn = (((0,), (0,)), ((), ()))
    psum = lax.dot_general(onehot, hn, dn,
                           preferred_element_type=jnp.float32,
                           precision=lax.Precision.HIGHEST)
    pcnt = lax.dot_general(onehot, jnp.ones_like(hn), dn,
                           preferred_element_type=jnp.float32,
                           precision=lax.Precision.HIGHEST)

    @pl.when(i == 0)
    def _():
        pool_acc[...] = psum
        cnt_acc[...] = pcnt

    @pl.when(i > 0)
    def _():
        pool_acc[...] += psum
        cnt_acc[...] += pcnt

    @pl.when(i == NB - 1)
    def _():
        pooled = pool_acc[...] / jnp.maximum(cnt_acc[...], 1.0)
        r = jnp.maximum(
            jnp.dot(pooled, w1_ref[...], preferred_element_type=jnp.float32)
            + b1_ref[...], 0.0)
        out_ref[...] = (
            jnp.dot(r, w2_ref[...], preferred_element_type=jnp.float32)
            + b2_ref[...])


def _make_pool(hdim, cdim):
    return pl.pallas_call(
        _pool_body,
        grid=(NB,),
        in_specs=[
            pl.BlockSpec((BN, hdim), lambda i: (i, 0)),
            pl.BlockSpec((1, hdim), lambda i: (0, 0)),
            pl.BlockSpec((1, hdim), lambda i: (0, 0)),
            pl.BlockSpec((1, hdim), lambda i: (0, 0)),
            pl.BlockSpec((1, hdim), lambda i: (0, 0)),
            pl.BlockSpec((1, BN, 1), lambda i: (i, 0, 0)),
            pl.BlockSpec((hdim, hdim), lambda i: (0, 0)),
            pl.BlockSpec((1, hdim), lambda i: (0, 0)),
            pl.BlockSpec((hdim, cdim), lambda i: (0, 0)),
            pl.BlockSpec((1, cdim), lambda i: (0, 0)),
        ],
        out_specs=[pl.BlockSpec((G, cdim), lambda i: (0, 0))],
        out_shape=[jax.ShapeDtypeStruct((G, cdim), jnp.float32)],
        scratch_shapes=[
            pltpu.VMEM((G, hdim), jnp.float32),
            pltpu.VMEM((G, hdim), jnp.float32),
        ],
    )


def kernel(x, params, edge_index, batch):
    n, f = x.shape
    e = edge_index.shape[1]
    layers = params["layers"]
    hdim = layers[0]["W1"].shape[1]
    cdim = params["lin2_W"].shape[1]

    src = edge_index[0].astype(jnp.int32)
    dst = edge_index[1].astype(jnp.int32)
    rt1 = -(-(-(-e // 4096)) // 40) * 40
    epad = rt1 * 4096
    pad = epad - e
    srcp = jnp.concatenate([src, jnp.zeros((pad,), jnp.int32)])
    dstp = jnp.concatenate([dst, jnp.full((pad,), DUMMY, jnp.int32)])
    src_l1 = srcp.reshape(epad // 128, 2, 64)
    src_l23 = jnp.stack([srcp, srcp + n]).reshape(2, epad // 128, 2, 64)
    dst2 = dstp.reshape(epad // 128, 2, 64)

    bat3 = batch.astype(jnp.int32).reshape(NB, BN, 1)

    agg1 = _sc_agg(x, src_l1, dst2, split=False)
    lp = layers[0]
    h2, ssum, ssq = _make_mlp(False, f, hdim)(
        lp["eps"].reshape(1), x, agg1, lp["W1"], lp["b1"].reshape(1, hdim),
        lp["W2"], lp["b2"].reshape(1, hdim))
    hcat = _make_bn_split(hdim)(
        h2, ssum, ssq, lp["gamma"].reshape(1, hdim),
        lp["beta"].reshape(1, hdim))[0]

    for li in range(1, len(layers)):
        lp = layers[li]
        table = hcat.reshape(2 * n, hdim // 2)
        agg = _sc_agg(table, src_l23, dst2, split=True)
        h2, ssum, ssq = _make_mlp(True, hdim, hdim)(
            lp["eps"].reshape(1), hcat, agg, lp["W1"],
            lp["b1"].reshape(1, hdim), lp["W2"], lp["b2"].reshape(1, hdim))
        if li < len(layers) - 1:
            hcat = _make_bn_split(hdim)(
                h2, ssum, ssq, lp["gamma"].reshape(1, hdim),
                lp["beta"].reshape(1, hdim))[0]

    lp = layers[-1]
    out = _make_pool(hdim, cdim)(
        h2, ssum, ssq, lp["gamma"].reshape(1, hdim),
        lp["beta"].reshape(1, hdim), bat3,
        params["lin1_W"], params["lin1_b"].reshape(1, hdim),
        params["lin2_W"], params["lin2_b"].reshape(1, cdim))[0]
    return out


# speedup vs baseline: 1.0462x; 1.0462x over previous
import functools

import jax
import jax.numpy as jnp
from jax import lax
from jax.experimental import pallas as pl
from jax.experimental.pallas import tpu as pltpu
from jax.experimental.pallas import tpu_sc as plsc

N = 10000
G = 64
NP = 10112
DUMMY = 10008
NSUB = 16
ZCH = 632
BN = 1000
NB = N // BN

def _make_sc_agg(rt: int, split: bool, er: int):
    mesh = plsc.VectorSubcoreMesh(core_axis_name="c", subcore_axis_name="s")

    @functools.partial(
        pl.kernel,
        out_type=jax.ShapeDtypeStruct((2, N, 128), jnp.float32),
        mesh=mesh,
        scratch_types=[
            pltpu.VMEM((40, 128), jnp.int32),
            pltpu.VMEM((40, 128), jnp.int32),
            pltpu.VMEM((128, 128), jnp.float32),
            pltpu.VMEM((128, 128), jnp.float32),
            pltpu.VMEM_SHARED((NP, 128), jnp.float32),
            pltpu.SemaphoreType.DMA,
            pltpu.SemaphoreType.DMA,
            pltpu.SemaphoreType.DMA,
            pltpu.SemaphoreType.DMA,
        ],
    )
    def sc_agg(h_hbm, src_hbm, dst_hbm, out_hbm, src_v, dst_v,
               rows_a, rows_b, acc, ga_sem, gb_sem, sa_sem, sb_sem):
        c = lax.axis_index("c")
        s = lax.axis_index("s")
        if split:
            base = s * rt
        else:
            base = (c * NSUB + s) * rt

        def zrow(i, carry):
            for j in range(8):
                rows_a[i, pl.ds(j * 16, 16)] = jnp.zeros((16,), jnp.float32)
            return carry

        lax.fori_loop(0, 128, zrow, 0)
        zoff = s * ZCH
        for k in range(4):
            pltpu.sync_copy(rows_a.at[pl.ds(0, 128)],
                            acc.at[pl.ds(zoff + 128 * k, 128)])
        pltpu.sync_copy(rows_a.at[pl.ds(0, ZCH - 512)],
                        acc.at[pl.ds(zoff + 512, ZCH - 512)])
        plsc.subcore_barrier()

        def chunk(ic, carry):
            cb = base + ic * 40
            if split:
                pltpu.sync_copy(src_hbm.at[c, pl.ds(cb, 40)], src_v)
            else:
                pltpu.sync_copy(src_hbm.at[pl.ds(cb, 40)], src_v)
            pltpu.sync_copy(dst_hbm.at[pl.ds(cb, 40)], dst_v)
            pltpu.async_copy(h_hbm.at[src_v.at[0]], rows_a, ga_sem)
            pltpu.async_copy(h_hbm.at[src_v.at[1]], rows_b, gb_sem)

            def pair(p, carry2):
                j0 = 2 * p
                pltpu.make_async_copy(h_hbm.at[src_v.at[j0]],
                                      rows_a, ga_sem).wait()
                pltpu.async_copy(rows_a, acc.at[dst_v.at[j0]], sa_sem,
                                 add=True)
                pltpu.make_async_copy(h_hbm.at[src_v.at[j0 + 1]],
                                      rows_b, gb_sem).wait()
                pltpu.async_copy(rows_b, acc.at[dst_v.at[j0 + 1]], sb_sem,
                                 add=True)

                @pl.when(p < 19)
                def _():
                    pltpu.make_async_copy(rows_a, acc.at[dst_v.at[0]],
                                          sa_sem).wait()
                    pltpu.async_copy(h_hbm.at[src_v.at[j0 + 2]],
                                     rows_a, ga_sem)
                    pltpu.make_async_copy(rows_b, acc.at[dst_v.at[0]],
                                          sb_sem).wait()
                    pltpu.async_copy(h_hbm.at[src_v.at[j0 + 3]],
                                     rows_b, gb_sem)
                return carry2

            lax.fori_loop(0, 20, pair, 0)
            pltpu.make_async_copy(rows_a, acc.at[dst_v.at[0]],
                                  sa_sem).wait()
            pltpu.make_async_copy(rows_b, acc.at[dst_v.at[0]],
                                  sb_sem).wait()
            return carry

        lax.fori_loop(0, rt // 40, chunk, 0)
        plsc.subcore_barrier()

        ooff = s * ZCH
        for k in range(4):
            pltpu.sync_copy(acc.at[pl.ds(ooff + 128 * k, 128)],
                            out_hbm.at[c, pl.ds(ooff + 128 * k, 128)])

        @pl.when(s < NSUB - 1)
        def _():
            pltpu.sync_copy(acc.at[pl.ds(ooff + 512, ZCH - 512)],
                            out_hbm.at[c, pl.ds(ooff + 512, ZCH - 512)])

        @pl.when(s == NSUB - 1)
        def _():
            rem = N - (NSUB - 1) * ZCH - 512
            pltpu.sync_copy(acc.at[pl.ds(ooff + 512, rem)],
                            out_hbm.at[c, pl.ds(ooff + 512, rem)])

    return sc_agg


_SC_AGG_CACHE = {}


def _sc_agg(table, src2, dst2, split):
    er = dst2.shape[0]
    key = (split, er)
    if key not in _SC_AGG_CACHE:
        rt = er // NSUB if split else er // (2 * NSUB)
        _SC_AGG_CACHE[key] = _make_sc_agg(rt, split, er)
    return _SC_AGG_CACHE[key](table, src2, dst2)


def _mlp_body(split, eps_ref, h_ref, agg_ref, w1_ref, b1_ref, w2_ref, b2_ref,
              h2_ref, ssum_ref, ssq_ref):
    i = pl.program_id(0)
    if split:
        h = jnp.concatenate([h_ref[0], h_ref[1]], axis=1)
        a = jnp.concatenate([agg_ref[0], agg_ref[1]], axis=1)
    else:
        h = h_ref[...]
        a = agg_ref[0] + agg_ref[1]
    z = (1.0 + eps_ref[0]) * h + a
    t = jnp.maximum(
        jnp.dot(z, w1_ref[...], preferred_element_type=jnp.float32)
        + b1_ref[...], 0.0)
    h2 = jnp.maximum(
        jnp.dot(t, w2_ref[...], preferred_element_type=jnp.float32)
        + b2_ref[...], 0.0)
    h2_ref[...] = h2
    ssum = jnp.sum(h2, axis=0, keepdims=True)
    ssq = jnp.sum(h2 * h2, axis=0, keepdims=True)

    @pl.when(i == 0)
    def _():
        ssum_ref[...] = ssum
        ssq_ref[...] = ssq

    @pl.when(i > 0)
    def _():
        ssum_ref[...] += ssum
        ssq_ref[...] += ssq


def _make_mlp(split, din, hdim):
    dh = din // 2
    if split:
        h_spec = pl.BlockSpec((2, BN, dh), lambda i: (0, i, 0))
    else:
        h_spec = pl.BlockSpec((BN, din), lambda i: (i, 0))
    return pl.pallas_call(
        functools.partial(_mlp_body, split),
        grid=(NB,),
        in_specs=[
            pl.BlockSpec(memory_space=pltpu.SMEM),
            h_spec,
            pl.BlockSpec((2, BN, dh if split else din), lambda i: (0, i, 0)),
            pl.BlockSpec((din, hdim), lambda i: (0, 0)),
            pl.BlockSpec((1, hdim), lambda i: (0, 0)),
            pl.BlockSpec((hdim, hdim), lambda i: (0, 0)),
            pl.BlockSpec((1, hdim), lambda i: (0, 0)),
        ],
        out_specs=[
            pl.BlockSpec((BN, hdim), lambda i: (i, 0)),
            pl.BlockSpec((1, hdim), lambda i: (0, 0)),
            pl.BlockSpec((1, hdim), lambda i: (0, 0)),
        ],
        out_shape=[
            jax.ShapeDtypeStruct((N, hdim), jnp.float32),
            jax.ShapeDtypeStruct((1, hdim), jnp.float32),
            jax.ShapeDtypeStruct((1, hdim), jnp.float32),
        ],
    )


def _bn_split_body(h2_ref, ssum_ref, ssq_ref, g_ref, b_ref, out_ref):
    mean = ssum_ref[...] * (1.0 / N)
    var = ssq_ref[...] * (1.0 / N) - mean * mean
    scale = g_ref[...] / jnp.sqrt(var + 1e-5)
    hn = (h2_ref[...] - mean) * scale + b_ref[...]
    dh = hn.shape[1] // 2
    out_ref[0] = hn[:, :dh]
    out_ref[1] = hn[:, dh:]


def _make_bn_split(hdim):
    dh = hdim // 2
    return pl.pallas_call(
        _bn_split_body,
        grid=(NB,),
        in_specs=[
            pl.BlockSpec((BN, hdim), lambda i: (i, 0)),
            pl.BlockSpec((1, hdim), lambda i: (0, 0)),
            pl.BlockSpec((1, hdim), lambda i: (0, 0)),
            pl.BlockSpec((1, hdim), lambda i: (0, 0)),
            pl.BlockSpec((1, hdim), lambda i: (0, 0)),
        ],
        out_specs=[pl.BlockSpec((2, BN, dh), lambda i: (0, i, 0))],
        out_shape=[jax.ShapeDtypeStruct((2, N, dh), jnp.float32)],
    )


def _pool_body(h2_ref, ssum_ref, ssq_ref, g_ref, b_ref, bat_ref,
               w1_ref, b1_ref, w2_ref, b2_ref, out_ref, pool_acc, cnt_acc):
    i = pl.program_id(0)
    mean = ssum_ref[...] * (1.0 / N)
    var = ssq_ref[...] * (1.0 / N) - mean * mean
    scale = g_ref[...] / jnp.sqrt(var + 1e-5)
    hn = (h2_ref[...] - mean) * scale + b_ref[...]
    bcol = bat_ref[0]
    onehot = (bcol == lax.broadcasted_iota(jnp.int32, (BN, G), 1)
              ).astype(jnp.float32)
    dn = (((0,), (0,)), ((), ()))
    psum = lax.dot_general(onehot, hn, dn,
                           preferred_element_type=jnp.float32,
                           precision=lax.Precision.HIGHEST)
    pcnt = lax.dot_general(onehot, jnp.ones_like(hn), dn,
                           preferred_element_type=jnp.float32,
                           precision=lax.Precision.HIGHEST)

    @pl.when(i == 0)
    def _():
        pool_acc[...] = psum
        cnt_acc[...] = pcnt

    @pl.when(i > 0)
    def _():
        pool_acc[...] += psum
        cnt_acc[...] += pcnt

    @pl.when(i == NB - 1)
    def _():
        pooled = pool_acc[...] / jnp.maximum(cnt_acc[...], 1.0)
        r = jnp.maximum(
            jnp.dot(pooled, w1_ref[...], preferred_element_type=jnp.float32)
            + b1_ref[...], 0.0)
        out_ref[...] = (
            jnp.dot(r, w2_ref[...], preferred_element_type=jnp.float32)
            + b2_ref[...])


def _make_pool(hdim, cdim):
    return pl.pallas_call(
        _pool_body,
        grid=(NB,),
        in_specs=[
            pl.BlockSpec((BN, hdim), lambda i: (i, 0)),
            pl.BlockSpec((1, hdim), lambda i: (0, 0)),
            pl.BlockSpec((1, hdim), lambda i: (0, 0)),
            pl.BlockSpec((1, hdim), lambda i: (0, 0)),
            pl.BlockSpec((1, hdim), lambda i: (0, 0)),
            pl.BlockSpec((1, BN, 1), lambda i: (i, 0, 0)),
            pl.BlockSpec((hdim, hdim), lambda i: (0, 0)),
            pl.BlockSpec((1, hdim), lambda i: (0, 0)),
            pl.BlockSpec((hdim, cdim), lambda i: (0, 0)),
            pl.BlockSpec((1, cdim), lambda i: (0, 0)),
        ],
        out_specs=[pl.BlockSpec((G, cdim), lambda i: (0, 0))],
        out_shape=[jax.ShapeDtypeStruct((G, cdim), jnp.float32)],
        scratch_shapes=[
            pltpu.VMEM((G, hdim), jnp.float32),
            pltpu.VMEM((G, hdim), jnp.float32),
        ],
    )


def kernel(x, params, edge_index, batch):
    n, f = x.shape
    e = edge_index.shape[1]
    layers = params["layers"]
    hdim = layers[0]["W1"].shape[1]
    cdim = params["lin2_W"].shape[1]

    src = edge_index[0].astype(jnp.int32)
    dst = edge_index[1].astype(jnp.int32)
    rt1 = -(-(-(-e // 4096)) // 40) * 40
    epad = rt1 * 4096
    pad = epad - e
    srcp = jnp.concatenate([src, jnp.zeros((pad,), jnp.int32)])
    dstp = jnp.concatenate([dst, jnp.full((pad,), DUMMY, jnp.int32)])
    src_l1 = srcp.reshape(epad // 128, 128)
    src_l23 = jnp.stack([srcp, srcp + n]).reshape(2, epad // 128, 128)
    dst2 = dstp.reshape(epad // 128, 128)

    bat3 = batch.astype(jnp.int32).reshape(NB, BN, 1)

    agg1 = _sc_agg(x, src_l1, dst2, split=False)
    lp = layers[0]
    h2, ssum, ssq = _make_mlp(False, f, hdim)(
        lp["eps"].reshape(1), x, agg1, lp["W1"], lp["b1"].reshape(1, hdim),
        lp["W2"], lp["b2"].reshape(1, hdim))
    hcat = _make_bn_split(hdim)(
        h2, ssum, ssq, lp["gamma"].reshape(1, hdim),
        lp["beta"].reshape(1, hdim))[0]

    for li in range(1, len(layers)):
        lp = layers[li]
        table = hcat.reshape(2 * n, hdim // 2)
        agg = _sc_agg(table, src_l23, dst2, split=True)
        h2, ssum, ssq = _make_mlp(True, hdim, hdim)(
            lp["eps"].reshape(1), hcat, agg, lp["W1"],
            lp["b1"].reshape(1, hdim), lp["W2"], lp["b2"].reshape(1, hdim))
        if li < len(layers) - 1:
            hcat = _make_bn_split(hdim)(
                h2, ssum, ssq, lp["gamma"].reshape(1, hdim),
                lp["beta"].reshape(1, hdim))[0]

    lp = layers[-1]
    out = _make_pool(hdim, cdim)(
        h2, ssum, ssq, lp["gamma"].reshape(1, hdim),
        lp["beta"].reshape(1, hdim), bat3,
        params["lin1_W"], params["lin1_b"].reshape(1, hdim),
        params["lin2_W"], params["lin2_b"].reshape(1, cdim))[0]
    return out


# speedup vs baseline: 1.1269x; 1.0771x over previous
import functools

import jax
import jax.numpy as jnp
from jax import lax
from jax.experimental import pallas as pl
from jax.experimental.pallas import tpu as pltpu
from jax.experimental.pallas import tpu_sc as plsc

N = 10000
G = 64
NP = 10112
DUMMY = 10008
NSUB = 16
ZCH = 632
BN = 1000
NB = N // BN

def _make_sc_agg(rt: int, split: bool, er: int):
    mesh = plsc.VectorSubcoreMesh(core_axis_name="c", subcore_axis_name="s")

    @functools.partial(
        pl.kernel,
        out_type=jax.ShapeDtypeStruct((2, N, 128), jnp.float32),
        mesh=mesh,
        scratch_types=[
            pltpu.VMEM((40, 128), jnp.int32),
            pltpu.VMEM((40, 128), jnp.int32),
            pltpu.VMEM((128, 128), jnp.float32),
            pltpu.VMEM((128, 128), jnp.float32),
            pltpu.VMEM_SHARED((NP, 128), jnp.float32),
            pltpu.SemaphoreType.DMA,
            pltpu.SemaphoreType.DMA,
        ],
    )
    def sc_agg(h_hbm, src_hbm, dst_hbm, out_hbm, src_v, dst_v,
               rows_a, rows_b, acc, sem_a, sem_b):
        c = lax.axis_index("c")
        s = lax.axis_index("s")
        if split:
            base = s * rt
        else:
            base = (c * NSUB + s) * rt

        def zrow(i, carry):
            for j in range(8):
                rows_a[i, pl.ds(j * 16, 16)] = jnp.zeros((16,), jnp.float32)
            return carry

        lax.fori_loop(0, 128, zrow, 0)
        zoff = s * ZCH
        for k in range(4):
            pltpu.sync_copy(rows_a.at[pl.ds(0, 128)],
                            acc.at[pl.ds(zoff + 128 * k, 128)])
        pltpu.sync_copy(rows_a.at[pl.ds(0, ZCH - 512)],
                        acc.at[pl.ds(zoff + 512, ZCH - 512)])
        plsc.subcore_barrier()

        def chunk(ic, carry):
            cb = base + ic * 40
            if split:
                pltpu.sync_copy(src_hbm.at[c, pl.ds(cb, 40)], src_v)
            else:
                pltpu.sync_copy(src_hbm.at[pl.ds(cb, 40)], src_v)
            pltpu.sync_copy(dst_hbm.at[pl.ds(cb, 40)], dst_v)
            pltpu.async_copy(h_hbm.at[src_v.at[0]], rows_a, sem_a)

            def pair(p, carry2):
                j0 = 2 * p
                pltpu.async_copy(h_hbm.at[src_v.at[j0 + 1]], rows_b, sem_b)
                pltpu.make_async_copy(h_hbm.at[src_v.at[j0]],
                                      rows_a, sem_a).wait()
                pltpu.sync_copy(rows_a, acc.at[dst_v.at[j0]], add=True)

                @pl.when(p < 19)
                def _():
                    pltpu.async_copy(h_hbm.at[src_v.at[j0 + 2]],
                                     rows_a, sem_a)

                pltpu.make_async_copy(h_hbm.at[src_v.at[j0 + 1]],
                                      rows_b, sem_b).wait()
                pltpu.sync_copy(rows_b, acc.at[dst_v.at[j0 + 1]], add=True)
                return carry2

            lax.fori_loop(0, 20, pair, 0)
            return carry

        lax.fori_loop(0, rt // 40, chunk, 0)
        plsc.subcore_barrier()

        ooff = s * ZCH
        for k in range(4):
            pltpu.sync_copy(acc.at[pl.ds(ooff + 128 * k, 128)],
                            out_hbm.at[c, pl.ds(ooff + 128 * k, 128)])

        @pl.when(s < NSUB - 1)
        def _():
            pltpu.sync_copy(acc.at[pl.ds(ooff + 512, ZCH - 512)],
                            out_hbm.at[c, pl.ds(ooff + 512, ZCH - 512)])

        @pl.when(s == NSUB - 1)
        def _():
            rem = N - (NSUB - 1) * ZCH - 512
            pltpu.sync_copy(acc.at[pl.ds(ooff + 512, rem)],
                            out_hbm.at[c, pl.ds(ooff + 512, rem)])

    return sc_agg


_SC_AGG_CACHE = {}


def _sc_agg(table, src2, dst2, split):
    er = dst2.shape[0]
    key = (split, er)
    if key not in _SC_AGG_CACHE:
        rt = er // NSUB if split else er // (2 * NSUB)
        _SC_AGG_CACHE[key] = _make_sc_agg(rt, split, er)
    return _SC_AGG_CACHE[key](table, src2, dst2)


def _mlp_body(split, eps_ref, h_ref, agg_ref, w1_ref, b1_ref, w2_ref, b2_ref,
              h2_ref, ssum_ref, ssq_ref):
    i = pl.program_id(0)
    if split:
        h = jnp.concatenate([h_ref[0], h_ref[1]], axis=1)
        a = jnp.concatenate([agg_ref[0], agg_ref[1]], axis=1)
    else:
        h = h_ref[...]
        a = agg_ref[0] + agg_ref[1]
    z = (1.0 + eps_ref[0]) * h + a
    t = jnp.maximum(
        jnp.dot(z, w1_ref[...], preferred_element_type=jnp.float32)
        + b1_ref[...], 0.0)
    h2 = jnp.maximum(
        jnp.dot(t, w2_ref[...], preferred_element_type=jnp.float32)
        + b2_ref[...], 0.0)
    h2_ref[...] = h2
    ssum = jnp.sum(h2, axis=0, keepdims=True)
    ssq = jnp.sum(h2 * h2, axis=0, keepdims=True)

    @pl.when(i == 0)
    def _():
        ssum_ref[...] = ssum
        ssq_ref[...] = ssq

    @pl.when(i > 0)
    def _():
        ssum_ref[...] += ssum
        ssq_ref[...] += ssq


def _make_mlp(split, din, hdim):
    dh = din // 2
    if split:
        h_spec = pl.BlockSpec((2, BN, dh), lambda i: (0, i, 0))
    else:
        h_spec = pl.BlockSpec((BN, din), lambda i: (i, 0))
    return pl.pallas_call(
        functools.partial(_mlp_body, split),
        grid=(NB,),
        in_specs=[
            pl.BlockSpec(memory_space=pltpu.SMEM),
            h_spec,
            pl.BlockSpec((2, BN, dh if split else din), lambda i: (0, i, 0)),
            pl.BlockSpec((din, hdim), lambda i: (0, 0)),
            pl.BlockSpec((1, hdim), lambda i: (0, 0)),
            pl.BlockSpec((hdim, hdim), lambda i: (0, 0)),
            pl.BlockSpec((1, hdim), lambda i: (0, 0)),
        ],
        out_specs=[
            pl.BlockSpec((BN, hdim), lambda i: (i, 0)),
            pl.BlockSpec((1, hdim), lambda i: (0, 0)),
            pl.BlockSpec((1, hdim), lambda i: (0, 0)),
        ],
        out_shape=[
            jax.ShapeDtypeStruct((N, hdim), jnp.float32),
            jax.ShapeDtypeStruct((1, hdim), jnp.float32),
            jax.ShapeDtypeStruct((1, hdim), jnp.float32),
        ],
    )


def _bn_split_body(h2_ref, ssum_ref, ssq_ref, g_ref, b_ref, out_ref):
    mean = ssum_ref[...] * (1.0 / N)
    var = ssq_ref[...] * (1.0 / N) - mean * mean
    scale = g_ref[...] / jnp.sqrt(var + 1e-5)
    hn = (h2_ref[...] - mean) * scale + b_ref[...]
    dh = hn.shape[1] // 2
    out_ref[0] = hn[:, :dh]
    out_ref[1] = hn[:, dh:]


def _make_bn_split(hdim):
    dh = hdim // 2
    return pl.pallas_call(
        _bn_split_body,
        grid=(NB,),
        in_specs=[
            pl.BlockSpec((BN, hdim), lambda i: (i, 0)),
            pl.BlockSpec((1, hdim), lambda i: (0, 0)),
            pl.BlockSpec((1, hdim), lambda i: (0, 0)),
            pl.BlockSpec((1, hdim), lambda i: (0, 0)),
            pl.BlockSpec((1, hdim), lambda i: (0, 0)),
        ],
        out_specs=[pl.BlockSpec((2, BN, dh), lambda i: (0, i, 0))],
        out_shape=[jax.ShapeDtypeStruct((2, N, dh), jnp.float32)],
    )


def _pool_body(h2_ref, ssum_ref, ssq_ref, g_ref, b_ref, bat_ref,
               w1_ref, b1_ref, w2_ref, b2_ref, out_ref, pool_acc, cnt_acc):
    i = pl.program_id(0)
    mean = ssum_ref[...] * (1.0 / N)
    var = ssq_ref[...] * (1.0 / N) - mean * mean
    scale = g_ref[...] / jnp.sqrt(var + 1e-5)
    hn = (h2_ref[...] - mean) * scale + b_ref[...]
    bcol = bat_ref[0]
    onehot = (bcol == lax.broadcasted_iota(jnp.int32, (BN, G), 1)
              ).astype(jnp.float32)
    dn = (((0,), (0,)), ((), ()))
    psum = lax.dot_general(onehot, hn, dn,
                           preferred_element_type=jnp.float32,
                           precision=lax.Precision.HIGHEST)
    pcnt = lax.dot_general(onehot, jnp.ones_like(hn), dn,
                           preferred_element_type=jnp.float32,
                           precision=lax.Precision.HIGHEST)

    @pl.when(i == 0)
    def _():
        pool_acc[...] = psum
        cnt_acc[...] = pcnt

    @pl.when(i > 0)
    def _():
        pool_acc[...] += psum
        cnt_acc[...] += pcnt

    @pl.when(i == NB - 1)
    def _():
        pooled = pool_acc[...] / jnp.maximum(cnt_acc[...], 1.0)
        r = jnp.maximum(
            jnp.dot(pooled, w1_ref[...], preferred_element_type=jnp.float32)
            + b1_ref[...], 0.0)
        out_ref[...] = (
            jnp.dot(r, w2_ref[...], preferred_element_type=jnp.float32)
            + b2_ref[...])


def _make_pool(hdim, cdim):
    return pl.pallas_call(
        _pool_body,
        grid=(NB,),
        in_specs=[
            pl.BlockSpec((BN, hdim), lambda i: (i, 0)),
            pl.BlockSpec((1, hdim), lambda i: (0, 0)),
            pl.BlockSpec((1, hdim), lambda i: (0, 0)),
            pl.BlockSpec((1, hdim), lambda i: (0, 0)),
            pl.BlockSpec((1, hdim), lambda i: (0, 0)),
            pl.BlockSpec((1, BN, 1), lambda i: (i, 0, 0)),
            pl.BlockSpec((hdim, hdim), lambda i: (0, 0)),
            pl.BlockSpec((1, hdim), lambda i: (0, 0)),
            pl.BlockSpec((hdim, cdim), lambda i: (0, 0)),
            pl.BlockSpec((1, cdim), lambda i: (0, 0)),
        ],
        out_specs=[pl.BlockSpec((G, cdim), lambda i: (0, 0))],
        out_shape=[jax.ShapeDtypeStruct((G, cdim), jnp.float32)],
        scratch_shapes=[
            pltpu.VMEM((G, hdim), jnp.float32),
            pltpu.VMEM((G, hdim), jnp.float32),
        ],
    )


def kernel(x, params, edge_index, batch):
    n, f = x.shape
    e = edge_index.shape[1]
    layers = params["layers"]
    hdim = layers[0]["W1"].shape[1]
    cdim = params["lin2_W"].shape[1]

    src = edge_index[0].astype(jnp.int32)
    dst = edge_index[1].astype(jnp.int32)
    rt1 = -(-(-(-e // 4096)) // 40) * 40
    epad = rt1 * 4096
    pad = epad - e
    srcp = jnp.concatenate([src, jnp.zeros((pad,), jnp.int32)])
    dstp = jnp.concatenate([dst, jnp.full((pad,), DUMMY, jnp.int32)])
    src_l1 = srcp.reshape(epad // 128, 128)
    src_l23 = jnp.stack([srcp, srcp + n]).reshape(2, epad // 128, 128)
    dst2 = dstp.reshape(epad // 128, 128)

    bat3 = batch.astype(jnp.int32).reshape(NB, BN, 1)

    agg1 = _sc_agg(x, src_l1, dst2, split=False)
    lp = layers[0]
    h2, ssum, ssq = _make_mlp(False, f, hdim)(
        lp["eps"].reshape(1), x, agg1, lp["W1"], lp["b1"].reshape(1, hdim),
        lp["W2"], lp["b2"].reshape(1, hdim))
    hcat = _make_bn_split(hdim)(
        h2, ssum, ssq, lp["gamma"].reshape(1, hdim),
        lp["beta"].reshape(1, hdim))[0]

    for li in range(1, len(layers)):
        lp = layers[li]
        table = hcat.reshape(2 * n, hdim // 2)
        agg = _sc_agg(table, src_l23, dst2, split=True)
        h2, ssum, ssq = _make_mlp(True, hdim, hdim)(
            lp["eps"].reshape(1), hcat, agg, lp["W1"],
            lp["b1"].reshape(1, hdim), lp["W2"], lp["b2"].reshape(1, hdim))
        if li < len(layers) - 1:
            hcat = _make_bn_split(hdim)(
                h2, ssum, ssq, lp["gamma"].reshape(1, hdim),
                lp["beta"].reshape(1, hdim))[0]

    lp = layers[-1]
    out = _make_pool(hdim, cdim)(
        h2, ssum, ssq, lp["gamma"].reshape(1, hdim),
        lp["beta"].reshape(1, hdim), bat3,
        params["lin1_W"], params["lin1_b"].reshape(1, hdim),
        params["lin2_W"], params["lin2_b"].reshape(1, cdim))[0]
    return out


# speedup vs baseline: 1.1361x; 1.0082x over previous
import functools

import jax
import jax.numpy as jnp
from jax import lax
from jax.experimental import pallas as pl
from jax.experimental.pallas import tpu as pltpu
from jax.experimental.pallas import tpu_sc as plsc

N = 10000
G = 64
NP = 10112
DUMMY = 10008
NSUB = 16
ZCH = 632
BN = 2000
NB = N // BN

def _make_sc_agg(rt: int, split: bool, er: int):
    mesh = plsc.VectorSubcoreMesh(core_axis_name="c", subcore_axis_name="s")

    @functools.partial(
        pl.kernel,
        out_type=jax.ShapeDtypeStruct((2, N, 128), jnp.float32),
        mesh=mesh,
        scratch_types=[
            pltpu.VMEM((40, 128), jnp.int32),
            pltpu.VMEM((40, 128), jnp.int32),
            pltpu.VMEM((128, 128), jnp.float32),
            pltpu.VMEM((128, 128), jnp.float32),
            pltpu.VMEM_SHARED((NP, 128), jnp.float32),
            pltpu.SemaphoreType.DMA,
            pltpu.SemaphoreType.DMA,
        ],
    )
    def sc_agg(h_hbm, src_hbm, dst_hbm, out_hbm, src_v, dst_v,
               rows_a, rows_b, acc, sem_a, sem_b):
        c = lax.axis_index("c")
        s = lax.axis_index("s")
        if split:
            base = s * rt
        else:
            base = (c * NSUB + s) * rt

        def zrow(i, carry):
            for j in range(8):
                rows_a[i, pl.ds(j * 16, 16)] = jnp.zeros((16,), jnp.float32)
            return carry

        lax.fori_loop(0, 128, zrow, 0)
        zoff = s * ZCH
        for k in range(4):
            pltpu.sync_copy(rows_a.at[pl.ds(0, 128)],
                            acc.at[pl.ds(zoff + 128 * k, 128)])
        pltpu.sync_copy(rows_a.at[pl.ds(0, ZCH - 512)],
                        acc.at[pl.ds(zoff + 512, ZCH - 512)])
        plsc.subcore_barrier()

        def chunk(ic, carry):
            cb = base + ic * 40
            if split:
                pltpu.sync_copy(src_hbm.at[c, pl.ds(cb, 40)], src_v)
            else:
                pltpu.sync_copy(src_hbm.at[pl.ds(cb, 40)], src_v)
            pltpu.sync_copy(dst_hbm.at[pl.ds(cb, 40)], dst_v)
            pltpu.async_copy(h_hbm.at[src_v.at[0]], rows_a, sem_a)

            def pair(p, carry2):
                j0 = 2 * p
                pltpu.async_copy(h_hbm.at[src_v.at[j0 + 1]], rows_b, sem_b)
                pltpu.make_async_copy(h_hbm.at[src_v.at[j0]],
                                      rows_a, sem_a).wait()
                pltpu.sync_copy(rows_a, acc.at[dst_v.at[j0]], add=True)

                @pl.when(p < 19)
                def _():
                    pltpu.async_copy(h_hbm.at[src_v.at[j0 + 2]],
                                     rows_a, sem_a)

                pltpu.make_async_copy(h_hbm.at[src_v.at[j0 + 1]],
                                      rows_b, sem_b).wait()
                pltpu.sync_copy(rows_b, acc.at[dst_v.at[j0 + 1]], add=True)
                return carry2

            lax.fori_loop(0, 20, pair, 0)
            return carry

        lax.fori_loop(0, rt // 40, chunk, 0)
        plsc.subcore_barrier()

        ooff = s * ZCH
        for k in range(4):
            pltpu.sync_copy(acc.at[pl.ds(ooff + 128 * k, 128)],
                            out_hbm.at[c, pl.ds(ooff + 128 * k, 128)])

        @pl.when(s < NSUB - 1)
        def _():
            pltpu.sync_copy(acc.at[pl.ds(ooff + 512, ZCH - 512)],
                            out_hbm.at[c, pl.ds(ooff + 512, ZCH - 512)])

        @pl.when(s == NSUB - 1)
        def _():
            rem = N - (NSUB - 1) * ZCH - 512
            pltpu.sync_copy(acc.at[pl.ds(ooff + 512, rem)],
                            out_hbm.at[c, pl.ds(ooff + 512, rem)])

    return sc_agg


_SC_AGG_CACHE = {}


def _sc_agg(table, src2, dst2, split):
    er = dst2.shape[0]
    key = (split, er)
    if key not in _SC_AGG_CACHE:
        rt = er // NSUB if split else er // (2 * NSUB)
        _SC_AGG_CACHE[key] = _make_sc_agg(rt, split, er)
    return _SC_AGG_CACHE[key](table, src2, dst2)


def _mlp_body(split, eps_ref, h_ref, agg_ref, w1_ref, b1_ref, w2_ref, b2_ref,
              h2_ref, ssum_ref, ssq_ref):
    i = pl.program_id(0)
    if split:
        h = jnp.concatenate([h_ref[0], h_ref[1]], axis=1)
        a = jnp.concatenate([agg_ref[0], agg_ref[1]], axis=1)
    else:
        h = h_ref[...]
        a = agg_ref[0] + agg_ref[1]
    z = (1.0 + eps_ref[0]) * h + a
    t = jnp.maximum(
        jnp.dot(z, w1_ref[...], preferred_element_type=jnp.float32)
        + b1_ref[...], 0.0)
    h2 = jnp.maximum(
        jnp.dot(t, w2_ref[...], preferred_element_type=jnp.float32)
        + b2_ref[...], 0.0)
    h2_ref[...] = h2
    ssum = jnp.sum(h2, axis=0, keepdims=True)
    ssq = jnp.sum(h2 * h2, axis=0, keepdims=True)

    @pl.when(i == 0)
    def _():
        ssum_ref[...] = ssum
        ssq_ref[...] = ssq

    @pl.when(i > 0)
    def _():
        ssum_ref[...] += ssum
        ssq_ref[...] += ssq


def _make_mlp(split, din, hdim):
    dh = din // 2
    if split:
        h_spec = pl.BlockSpec((2, BN, dh), lambda i: (0, i, 0))
    else:
        h_spec = pl.BlockSpec((BN, din), lambda i: (i, 0))
    return pl.pallas_call(
        functools.partial(_mlp_body, split),
        grid=(NB,),
        in_specs=[
            pl.BlockSpec(memory_space=pltpu.SMEM),
            h_spec,
            pl.BlockSpec((2, BN, dh if split else din), lambda i: (0, i, 0)),
            pl.BlockSpec((din, hdim), lambda i: (0, 0)),
            pl.BlockSpec((1, hdim), lambda i: (0, 0)),
            pl.BlockSpec((hdim, hdim), lambda i: (0, 0)),
            pl.BlockSpec((1, hdim), lambda i: (0, 0)),
        ],
        out_specs=[
            pl.BlockSpec((BN, hdim), lambda i: (i, 0)),
            pl.BlockSpec((1, hdim), lambda i: (0, 0)),
            pl.BlockSpec((1, hdim), lambda i: (0, 0)),
        ],
        out_shape=[
            jax.ShapeDtypeStruct((N, hdim), jnp.float32),
            jax.ShapeDtypeStruct((1, hdim), jnp.float32),
            jax.ShapeDtypeStruct((1, hdim), jnp.float32),
        ],
    )


def _bn_split_body(h2_ref, ssum_ref, ssq_ref, g_ref, b_ref, out_ref):
    mean = ssum_ref[...] * (1.0 / N)
    var = ssq_ref[...] * (1.0 / N) - mean * mean
    scale = g_ref[...] / jnp.sqrt(var + 1e-5)
    hn = (h2_ref[...] - mean) * scale + b_ref[...]
    dh = hn.shape[1] // 2
    out_ref[0] = hn[:, :dh]
    out_ref[1] = hn[:, dh:]


def _make_bn_split(hdim):
    dh = hdim // 2
    return pl.pallas_call(
        _bn_split_body,
        grid=(NB,),
        in_specs=[
            pl.BlockSpec((BN, hdim), lambda i: (i, 0)),
            pl.BlockSpec((1, hdim), lambda i: (0, 0)),
            pl.BlockSpec((1, hdim), lambda i: (0, 0)),
            pl.BlockSpec((1, hdim), lambda i: (0, 0)),
            pl.BlockSpec((1, hdim), lambda i: (0, 0)),
        ],
        out_specs=[pl.BlockSpec((2, BN, dh), lambda i: (0, i, 0))],
        out_shape=[jax.ShapeDtypeStruct((2, N, dh), jnp.float32)],
    )


def _pool_body(h2_ref, ssum_ref, ssq_ref, g_ref, b_ref, bat_ref,
               w1_ref, b1_ref, w2_ref, b2_ref, out_ref, pool_acc, cnt_acc):
    i = pl.program_id(0)
    mean = ssum_ref[...] * (1.0 / N)
    var = ssq_ref[...] * (1.0 / N) - mean * mean
    scale = g_ref[...] / jnp.sqrt(var + 1e-5)
    hn = (h2_ref[...] - mean) * scale + b_ref[...]
    bcol = bat_ref[0]
    onehot = (bcol == lax.broadcasted_iota(jnp.int32, (BN, G), 1)
              ).astype(jnp.float32)
    dn = (((0,), (0,)), ((), ()))
    psum = lax.dot_general(onehot, hn, dn,
                           preferred_element_type=jnp.float32,
                           precision=lax.Precision.HIGHEST)
    pcnt = lax.dot_general(onehot, jnp.ones_like(hn), dn,
                           preferred_element_type=jnp.float32,
                           precision=lax.Precision.HIGHEST)

    @pl.when(i == 0)
    def _():
        pool_acc[...] = psum
        cnt_acc[...] = pcnt

    @pl.when(i > 0)
    def _():
        pool_acc[...] += psum
        cnt_acc[...] += pcnt

    @pl.when(i == NB - 1)
    def _():
        pooled = pool_acc[...] / jnp.maximum(cnt_acc[...], 1.0)
        r = jnp.maximum(
            jnp.dot(pooled, w1_ref[...], preferred_element_type=jnp.float32)
            + b1_ref[...], 0.0)
        out_ref[...] = (
            jnp.dot(r, w2_ref[...], preferred_element_type=jnp.float32)
            + b2_ref[...])


def _make_pool(hdim, cdim):
    return pl.pallas_call(
        _pool_body,
        grid=(NB,),
        in_specs=[
            pl.BlockSpec((BN, hdim), lambda i: (i, 0)),
            pl.BlockSpec((1, hdim), lambda i: (0, 0)),
            pl.BlockSpec((1, hdim), lambda i: (0, 0)),
            pl.BlockSpec((1, hdim), lambda i: (0, 0)),
            pl.BlockSpec((1, hdim), lambda i: (0, 0)),
            pl.BlockSpec((1, BN, 1), lambda i: (i, 0, 0)),
            pl.BlockSpec((hdim, hdim), lambda i: (0, 0)),
            pl.BlockSpec((1, hdim), lambda i: (0, 0)),
            pl.BlockSpec((hdim, cdim), lambda i: (0, 0)),
            pl.BlockSpec((1, cdim), lambda i: (0, 0)),
        ],
        out_specs=[pl.BlockSpec((G, cdim), lambda i: (0, 0))],
        out_shape=[jax.ShapeDtypeStruct((G, cdim), jnp.float32)],
        scratch_shapes=[
            pltpu.VMEM((G, hdim), jnp.float32),
            pltpu.VMEM((G, hdim), jnp.float32),
        ],
    )


def kernel(x, params, edge_index, batch):
    n, f = x.shape
    e = edge_index.shape[1]
    layers = params["layers"]
    hdim = layers[0]["W1"].shape[1]
    cdim = params["lin2_W"].shape[1]

    src = edge_index[0].astype(jnp.int32)
    dst = edge_index[1].astype(jnp.int32)
    rt1 = -(-(-(-e // 4096)) // 40) * 40
    epad = rt1 * 4096
    pad = epad - e
    srcp = jnp.concatenate([src, jnp.zeros((pad,), jnp.int32)])
    dstp = jnp.concatenate([dst, jnp.full((pad,), DUMMY, jnp.int32)])
    src_l1 = srcp.reshape(epad // 128, 128)
    src_l23 = jnp.stack([srcp, srcp + n]).reshape(2, epad // 128, 128)
    dst2 = dstp.reshape(epad // 128, 128)

    bat3 = batch.astype(jnp.int32).reshape(NB, BN, 1)

    agg1 = _sc_agg(x, src_l1, dst2, split=False)
    lp = layers[0]
    h2, ssum, ssq = _make_mlp(False, f, hdim)(
        lp["eps"].reshape(1), x, agg1, lp["W1"], lp["b1"].reshape(1, hdim),
        lp["W2"], lp["b2"].reshape(1, hdim))
    hcat = _make_bn_split(hdim)(
        h2, ssum, ssq, lp["gamma"].reshape(1, hdim),
        lp["beta"].reshape(1, hdim))[0]

    for li in range(1, len(layers)):
        lp = layers[li]
        table = hcat.reshape(2 * n, hdim // 2)
        agg = _sc_agg(table, src_l23, dst2, split=True)
        h2, ssum, ssq = _make_mlp(True, hdim, hdim)(
            lp["eps"].reshape(1), hcat, agg, lp["W1"],
            lp["b1"].reshape(1, hdim), lp["W2"], lp["b2"].reshape(1, hdim))
        if li < len(layers) - 1:
            hcat = _make_bn_split(hdim)(
                h2, ssum, ssq, lp["gamma"].reshape(1, hdim),
                lp["beta"].reshape(1, hdim))[0]

    lp = layers[-1]
    out = _make_pool(hdim, cdim)(
        h2, ssum, ssq, lp["gamma"].reshape(1, hdim),
        lp["beta"].reshape(1, hdim), bat3,
        params["lin1_W"], params["lin1_b"].reshape(1, hdim),
        params["lin2_W"], params["lin2_b"].reshape(1, cdim))[0]
    return out
